# Initial kernel scaffold; baseline (speedup 1.0000x reference)
import jax
import jax.numpy as jnp
from jax.experimental import pallas as pl


def kernel(clear_feature, rain_feature, W1, b1, W2, b2):
    raise NotImplementedError("write your pallas kernel here")


# speedup vs baseline: 6.6304x; 6.6304x over previous
import functools

import jax
import jax.numpy as jnp
from jax import lax
from jax.experimental import pallas as pl
from jax.experimental.pallas import tpu as pltpu
from jax.experimental.pallas import tpu_sc as plsc

N_CLEAR = 4096
N_RAIN = 8192
D = 512

BI = 1024
BJ = 1024
BM = 1024

_SC_CORES = 2
_SC_SUBCORES = 16
_SC_WORKERS = _SC_CORES * _SC_SUBCORES
_ROWS_PER_WORKER = N_CLEAR // _SC_WORKERS


def _argmin_body(x_ref, y_ref, idx_ref, bv_ref, bi_ref):
    j = pl.program_id(1)
    nj = pl.num_programs(1)

    @pl.when(j == 0)
    def _init():
        bv_ref[...] = jnp.full((BI, 1), jnp.inf, jnp.float32)
        bi_ref[...] = jnp.zeros((BI, 1), jnp.int32)

    x = x_ref[...]
    y = y_ref[...]
    x2 = jnp.sum(x * x, axis=1, keepdims=True)
    y2 = jnp.sum(y * y, axis=1)[None, :]
    dot = lax.dot_general(x, y, (((1,), (1,)), ((), ())),
                          preferred_element_type=jnp.float32)
    sq = x2 + y2 - 2.0 * dot
    dist = jnp.sqrt(jnp.maximum(sq, 0.0))

    minv = jnp.min(dist, axis=1, keepdims=True)
    cols = lax.broadcasted_iota(jnp.int32, (BI, BJ), 1) + j * BJ
    lidx = jnp.min(jnp.where(dist == minv, cols, jnp.int32(2**30)),
                   axis=1, keepdims=True)

    better = minv < bv_ref[...]
    bv_ref[...] = jnp.where(better, minv, bv_ref[...])
    bi_ref[...] = jnp.where(better, lidx, bi_ref[...])

    @pl.when(j == nj - 1)
    def _emit():
        idx_ref[...] = bi_ref[...]


def _nearest_idx(clear, rain):
    grid = (N_CLEAR // BI, N_RAIN // BJ)
    return pl.pallas_call(
        _argmin_body,
        grid=grid,
        in_specs=[
            pl.BlockSpec((BI, D), lambda i, j: (i, 0)),
            pl.BlockSpec((BJ, D), lambda i, j: (j, 0)),
        ],
        out_specs=pl.BlockSpec((BI, 1), lambda i, j: (i, 0)),
        out_shape=jax.ShapeDtypeStruct((N_CLEAR, 1), jnp.int32),
        scratch_shapes=[
            pltpu.VMEM((BI, 1), jnp.float32),
            pltpu.VMEM((BI, 1), jnp.int32),
        ],
    )(clear, rain)


@functools.partial(
    pl.kernel,
    mesh=plsc.VectorSubcoreMesh(core_axis_name="c", subcore_axis_name="s"),
    out_type=jax.ShapeDtypeStruct((N_CLEAR, D), jnp.float32),
    scratch_types=[
        pltpu.VMEM((_ROWS_PER_WORKER,), jnp.int32),
        pltpu.VMEM((_ROWS_PER_WORKER, D), jnp.float32),
        pltpu.SemaphoreType.DMA,
    ],
)
def _sc_gather(table_hbm, idx_hbm, out_hbm, idx_v, rows_v, sem):
    wid = lax.axis_index("s") * _SC_CORES + lax.axis_index("c")
    base = wid * _ROWS_PER_WORKER
    pltpu.sync_copy(idx_hbm.at[pl.ds(base, _ROWS_PER_WORKER)], idx_v)
    pltpu.async_copy(table_hbm.at[idx_v], rows_v, sem).wait()
    pltpu.sync_copy(rows_v, out_hbm.at[pl.ds(base, _ROWS_PER_WORKER)])


def _mlp_body(x_ref, a_ref, w1_ref, b1_ref, w2_ref, b2_ref, out_ref):
    x = x_ref[...]
    a = a_ref[...]
    comb = jnp.concatenate([x, a], axis=1)
    h = jax.nn.relu(lax.dot_general(comb, w1_ref[...],
                                    (((1,), (0,)), ((), ())),
                                    preferred_element_type=jnp.float32)
                    + b1_ref[...])
    s = lax.dot_general(h, w2_ref[...], (((1,), (0,)), ((), ())),
                        preferred_element_type=jnp.float32) + b2_ref[...]
    w = jax.nn.sigmoid(s)
    out_ref[...] = w * x + (1.0 - w) * a


def _mlp_fuse(clear, aligned, W1, b1, W2, b2):
    grid = (N_CLEAR // BM,)
    return pl.pallas_call(
        _mlp_body,
        grid=grid,
        in_specs=[
            pl.BlockSpec((BM, D), lambda i: (i, 0)),
            pl.BlockSpec((BM, D), lambda i: (i, 0)),
            pl.BlockSpec((2 * D, D), lambda i: (0, 0)),
            pl.BlockSpec((1, D), lambda i: (0, 0)),
            pl.BlockSpec((D, 1), lambda i: (0, 0)),
            pl.BlockSpec((1, 1), lambda i: (0, 0)),
        ],
        out_specs=pl.BlockSpec((BM, D), lambda i: (i, 0)),
        out_shape=jax.ShapeDtypeStruct((N_CLEAR, D), jnp.float32),
    )(clear, aligned, W1, b1.reshape(1, D), W2, b2.reshape(1, 1))


def kernel(clear_feature, rain_feature, W1, b1, W2, b2):
    idx = _nearest_idx(clear_feature, rain_feature).reshape(N_CLEAR)
    aligned = _sc_gather(rain_feature, idx)
    return _mlp_fuse(clear_feature, aligned, W1, b1, W2, b2)
